# baseline (speedup 1.0000x reference)
import functools

import jax
import jax.numpy as jnp
from jax import lax
from jax.experimental import pallas as pl
from jax.experimental.pallas import tpu as pltpu
from jax.experimental.pallas import tpu_sc as plsc

N = 10000
E = 320000
D = 128
H = 128
HH = 64
V = 500
NF = 2048
G = 1024

NC = 2
NS = 16
K = 80
C = 800
NP = 10240
EPT = E // NS
NCH = EPT // K
RPT = NP // NS
FPT = NF // NS
GPW = G // (NC * NS)
NRPT = G // NS


def _tc_pre_body(x_ref, w_ref, b_ref, fw_ref, fb_ref, fr_ref, h_ref, fs_ref):
    i = pl.program_id(0)
    h = jnp.dot(x_ref[...], w_ref[...], preferred_element_type=jnp.float32)
    h = h + b_ref[...]
    h_ref[...] = h
    fp = jnp.sum(h * fw_ref[...], axis=1, keepdims=True) + fb_ref[0, 0]
    t = fr_ref[...]
    bce = jnp.maximum(fp, 0.0) - fp * t + jnp.log1p(jnp.exp(-jnp.abs(fp)))
    part = jnp.sum(bce)

    @pl.when(i == 0)
    def _():
        fs_ref[...] = jnp.zeros((1, 1), jnp.float32)

    fs_ref[...] += jnp.full((1, 1), part)


def _tc_pre(x, w, b, fw, fb, fr):
    nb = 10
    rb = N // nb
    return pl.pallas_call(
        _tc_pre_body,
        grid=(nb,),
        in_specs=[
            pl.BlockSpec((rb, D), lambda i: (i, 0)),
            pl.BlockSpec((D, H), lambda i: (0, 0)),
            pl.BlockSpec((1, H), lambda i: (0, 0)),
            pl.BlockSpec((1, H), lambda i: (0, 0)),
            pl.BlockSpec((1, 1), lambda i: (0, 0)),
            pl.BlockSpec((rb, 1), lambda i: (i, 0)),
        ],
        out_specs=[
            pl.BlockSpec((rb, H), lambda i: (i, 0)),
            pl.BlockSpec((1, 1), lambda i: (0, 0)),
        ],
        out_shape=[
            jax.ShapeDtypeStruct((N, H), jnp.float32),
            jax.ShapeDtypeStruct((1, 1), jnp.float32),
        ],
    )(x, w, b, fw, fb, fr)


WW = 32
WEPT = E // WW
WCH = 2000
WNCH = WEPT // WCH


def _sc_w_body(px, py, pz, qx, qy, qz, srch, dsth, wc_o, wn_o,
               pxv, pyv, pzv, qxv, qyv, qzv, srci, dsti, wcv, wnv):
    c = lax.axis_index("c")
    s = lax.axis_index("s")
    wid = c * NS + s
    pltpu.sync_copy(px, pxv)
    pltpu.sync_copy(py, pyv)
    pltpu.sync_copy(pz, pzv)
    pltpu.sync_copy(qx, qxv)
    pltpu.sync_copy(qy, qyv)
    pltpu.sync_copy(qz, qzv)
    ebase = wid * WEPT

    def chunk_body(i, carry):
        base = ebase + i * WCH
        pltpu.sync_copy(srch.at[pl.ds(base, WCH)], srci)
        pltpu.sync_copy(dsth.at[pl.ds(base, WCH)], dsti)

        def grp(g, cc):
            sl = pl.ds(g * 16, 16)
            s16 = srci[sl]
            d16 = dsti[sl]
            ax = plsc.load_gather(pxv, [s16]) - plsc.load_gather(pxv, [d16])
            ay = plsc.load_gather(pyv, [s16]) - plsc.load_gather(pyv, [d16])
            az = plsc.load_gather(pzv, [s16]) - plsc.load_gather(pzv, [d16])
            wcv[sl] = jnp.exp(-(ax * ax + ay * ay + az * az))
            bx = plsc.load_gather(qxv, [s16]) - plsc.load_gather(qxv, [d16])
            by = plsc.load_gather(qyv, [s16]) - plsc.load_gather(qyv, [d16])
            bz = plsc.load_gather(qzv, [s16]) - plsc.load_gather(qzv, [d16])
            wnv[sl] = jnp.exp(-(bx * bx + by * by + bz * bz))
            return cc

        lax.fori_loop(0, WCH // 16, grp, 0)
        pltpu.sync_copy(wcv, wc_o.at[pl.ds(base, WCH)])
        pltpu.sync_copy(wnv, wn_o.at[pl.ds(base, WCH)])
        return carry

    lax.fori_loop(0, WNCH, chunk_body, 0)


def _sc_weights(px, py, pz, qx, qy, qz, src, dst):
    f32 = jnp.float32
    i32 = jnp.int32
    mesh = plsc.VectorSubcoreMesh(core_axis_name="c", subcore_axis_name="s",
                                  num_cores=NC, num_subcores=NS)
    kfn = pl.kernel(
        _sc_w_body,
        out_type=[
            jax.ShapeDtypeStruct((E,), f32),
            jax.ShapeDtypeStruct((E,), f32),
        ],
        mesh=mesh,
        scratch_types=[
            pltpu.VMEM((N,), f32), pltpu.VMEM((N,), f32), pltpu.VMEM((N,), f32),
            pltpu.VMEM((N,), f32), pltpu.VMEM((N,), f32), pltpu.VMEM((N,), f32),
            pltpu.VMEM((WCH,), i32), pltpu.VMEM((WCH,), i32),
            pltpu.VMEM((WCH,), f32), pltpu.VMEM((WCH,), f32),
        ],
        compiler_params=pltpu.CompilerParams(needs_layout_passes=False,
                                             use_tc_tiling_on_sc=False),
    )
    return kfn(px, py, pz, qx, qy, qz, src, dst)


def _sc_body(hpk, srch, dsth, wch, wnh, cah, cabh, cwh, embh, zeros,
             aggc_o, aggn_o, nhpk_o, embsel_o,
             acc_c, acc_n, nh_acc,
             srci, dsti, dsti2, srcg, wcv, wnv,
             rows0, rows1, mc0, mc1, mn0, mn1,
             cav, cagv, cab2v, cwv, erow,
             gsem0, gsem1, scsem0, scsem1, snsem0, snsem1):
    c = lax.axis_index("c")
    s = lax.axis_index("s")
    cN = c * N

    pltpu.sync_copy(zeros, acc_c.at[pl.ds(s * RPT, RPT)])
    pltpu.sync_copy(zeros, acc_n.at[pl.ds(s * RPT, RPT)])
    pltpu.sync_copy(zeros.at[pl.ds(0, NRPT)], nh_acc.at[pl.ds(s * NRPT, NRPT)])
    plsc.subcore_barrier()

    ebase = s * EPT
    NSUB = C // K
    rbufs = (rows0, rows1)
    mbufs = ((mc0, mn0), (mc1, mn1))
    gsems = (gsem0, gsem1)
    ssems = ((scsem0, snsem0), (scsem1, snsem1))

    def chunk_body(i, carry):
        base = ebase + i * C
        pltpu.sync_copy(srch.at[pl.ds(base, C)], srci)
        pltpu.sync_copy(dsth.at[pl.ds(base, C)], dsti)
        pltpu.sync_copy(wch.at[pl.ds(base, C)], wcv)
        pltpu.sync_copy(wnh.at[pl.ds(base, C)], wnv)

        def prep(g, cc):
            sl = pl.ds(g * 16, 16)
            srcg[sl] = srci[sl] + cN
            return cc

        lax.fori_loop(0, C // 16, prep, 0)
        for r in range(NSUB):
            for g in range(K // 16):
                dsti2[r, pl.ds(g * 16, 16)] = dsti[pl.ds(r * K + g * 16, 16)]

        gd = pltpu.async_copy(hpk.at[srcg.at[pl.ds(0, K)]], rows0, gsem0)
        pend = []
        for j in range(NSUB):
            rb = rbufs[j % 2]
            mcb, mnb = mbufs[j % 2]
            gd_next = None
            if j + 1 < NSUB:
                gd_next = pltpu.async_copy(
                    hpk.at[srcg.at[pl.ds((j + 1) * K, K)]],
                    rbufs[(j + 1) % 2], gsems[(j + 1) % 2])
            gd.wait()
            if gd_next is not None:
                gd = gd_next
            if j >= 2:
                so, sn = pend[j - 2]
                so.wait()
                sn.wait()
            woff = j * K

            @plsc.parallel_loop(0, K // 16)
            def scale(g):
                w16c = wcv[pl.ds(woff + g * 16, 16)]
                w16n = wnv[pl.ds(woff + g * 16, 16)]
                for l in range(16):
                    e = g * 16 + l
                    wc_s = w16c[l]
                    wn_s = w16n[l]
                    for q in range(HH // 16):
                        sl = pl.ds(q * 16, 16)
                        r = rb[e, sl]
                        mcb[e, sl] = r * wc_s
                        mnb[e, sl] = r * wn_s
            so = pltpu.async_copy(mcb, acc_c.at[dsti2.at[j]],
                                  ssems[j % 2][0], add=True)
            sn = pltpu.async_copy(mnb, acc_n.at[dsti2.at[j]],
                                  ssems[j % 2][1], add=True)
            pend.append((so, sn))
        for so, sn in pend[-2:]:
            so.wait()
            sn.wait()
        return carry

    lax.fori_loop(0, EPT // C, chunk_body, 0)
    plsc.subcore_barrier()

    cNP = c * NP
    pltpu.sync_copy(acc_c.at[pl.ds(s * RPT, RPT)],
                    aggc_o.at[pl.ds(cNP + s * RPT, RPT)])
    pltpu.sync_copy(acc_n.at[pl.ds(s * RPT, RPT)],
                    aggn_o.at[pl.ds(cNP + s * RPT, RPT)])

    pltpu.sync_copy(cah.at[pl.ds(s * FPT, FPT)], cav)
    for g in range(FPT // 16):
        sl = pl.ds(g * 16, 16)
        cagv[sl] = cav[sl] + cN
    pltpu.sync_copy(cabh.at[pl.ds(s * FPT, FPT)], srci.at[pl.ds(0, FPT)])
    for r in range(FPT // 64):
        for g in range(64 // 16):
            cab2v[r, pl.ds(g * 16, 16)] = srci[pl.ds(r * 64 + g * 16, 16)]
    for r in range(FPT // 64):
        ha = rows0.at[pl.ds(0, 64)]
        ga = rows1.at[pl.ds(0, 64)]
        d1 = pltpu.async_copy(hpk.at[cagv.at[pl.ds(r * 64, 64)]], ha, gsem0)
        d2 = pltpu.async_copy(acc_c.at[cav.at[pl.ds(r * 64, 64)]], ga, gsem1)
        d1.wait()
        d2.wait()
        pltpu.sync_copy(ha, nh_acc.at[cab2v.at[r]], add=True)
        pltpu.sync_copy(ga, nh_acc.at[cab2v.at[r]], add=True)
    gbase = c * (G // NC) + s * GPW
    pltpu.sync_copy(cwh.at[pl.ds(gbase, GPW)], cwv)
    pltpu.async_copy(embh.at[cwv], erow, gsem0).wait()
    pltpu.sync_copy(erow, embsel_o.at[pl.ds(gbase, GPW)])
    plsc.subcore_barrier()

    pltpu.sync_copy(nh_acc.at[pl.ds(s * NRPT, NRPT)],
                    nhpk_o.at[pl.ds(c * G + s * NRPT, NRPT)])


def _sc_edges(hpk, src, dst, wc, wn, ca, cab, cw, emb, zeros):
    f32 = jnp.float32
    i32 = jnp.int32
    mesh = plsc.VectorSubcoreMesh(core_axis_name="c", subcore_axis_name="s",
                                  num_cores=NC, num_subcores=NS)
    kfn = pl.kernel(
        _sc_body,
        out_type=[
            jax.ShapeDtypeStruct((2 * NP, HH), f32),
            jax.ShapeDtypeStruct((2 * NP, HH), f32),
            jax.ShapeDtypeStruct((2 * G, HH), f32),
            jax.ShapeDtypeStruct((G, H), f32),
        ],
        mesh=mesh,
        scratch_types=[
            pltpu.VMEM_SHARED((NP, HH), f32),
            pltpu.VMEM_SHARED((NP, HH), f32),
            pltpu.VMEM_SHARED((G, HH), f32),
            pltpu.VMEM((C,), i32), pltpu.VMEM((C,), i32),
            pltpu.VMEM((C // K, K), i32), pltpu.VMEM((C,), i32),
            pltpu.VMEM((C,), f32), pltpu.VMEM((C,), f32),
            pltpu.VMEM((K, HH), f32), pltpu.VMEM((K, HH), f32),
            pltpu.VMEM((K, HH), f32), pltpu.VMEM((K, HH), f32),
            pltpu.VMEM((K, HH), f32), pltpu.VMEM((K, HH), f32),
            pltpu.VMEM((FPT,), i32), pltpu.VMEM((FPT,), i32),
            pltpu.VMEM((FPT // 64, 64), i32), pltpu.VMEM((GPW,), i32),
            pltpu.VMEM((GPW, H), f32),
            pltpu.SemaphoreType.DMA, pltpu.SemaphoreType.DMA,
            pltpu.SemaphoreType.DMA, pltpu.SemaphoreType.DMA,
            pltpu.SemaphoreType.DMA, pltpu.SemaphoreType.DMA,
        ],
        compiler_params=pltpu.CompilerParams(needs_layout_passes=False,
                                             use_tc_tiling_on_sc=False),
    )
    return kfn(hpk, src, dst, wc, wn, ca, cab, cw, emb, zeros)


def _tc_post_body(hl_ref, agc_ref, agn_ref, hlr_ref, agnr_ref,
                  nh_ref, emb_ref, w1_ref, w2_ref, wb_ref, wo_ref, wob_ref,
                  nw_ref, fs_ref, loss_ref, sacc_ref):
    i = pl.program_id(0)
    nb = pl.num_programs(0)

    @pl.when(i == 0)
    def _():
        sacc_ref[0] = 0.0
        sacc_ref[1] = 0.0

    hl = hl_ref[...]
    h = hl + agc_ref[...]
    h2 = hl + agn_ref[...]
    h2n = hlr_ref[...] + agnr_ref[...]

    def norm(x):
        n = jnp.sqrt(jnp.sum(x * x, axis=1, keepdims=True))
        return x / jnp.maximum(n, 1e-12)

    hn = norm(h)
    hn2 = norm(h2)
    hn2n = norm(h2n)
    pp = jnp.sum(hn * hn2, axis=1, keepdims=True)
    pn = jnp.sum(hn * hn2n, axis=1, keepdims=True)
    sp = jnp.log1p(jnp.exp(-jnp.abs(pp)))
    sacc_ref[0] += jnp.sum(jnp.maximum(pp, 0.0) - pp + sp)
    sn = jnp.log1p(jnp.exp(-jnp.abs(pn)))
    sacc_ref[1] += jnp.sum(jnp.maximum(pn, 0.0) + sn)

    @pl.when(i == nb - 1)
    def _():
        ssl_loss = (sacc_ref[0] / N + sacc_ref[1] / N) * 0.5
        pv = jnp.dot(nh_ref[...], w1_ref[...],
                     preferred_element_type=jnp.float32)
        pv = pv + jnp.dot(emb_ref[...], w2_ref[...],
                          preferred_element_type=jnp.float32)
        pv = jnp.maximum(pv + wb_ref[...], 0.0)
        scores = jnp.dot(pv, wo_ref[...],
                         preferred_element_type=jnp.float32) + wob_ref[...]
        m = jnp.max(scores, axis=1, keepdims=True)
        lz = jnp.log(jnp.sum(jnp.exp(scores - m), axis=1, keepdims=True)) + m
        cols = lax.broadcasted_iota(jnp.int32, scores.shape, 1)
        onehot = cols == nw_ref[...]
        tgt = jnp.sum(jnp.where(onehot, scores, 0.0), axis=1, keepdims=True)
        pred_loss = jnp.sum(lz - tgt) / G
        focal_loss = fs_ref[...][0, 0] / N
        loss_ref[...] = jnp.full((1, 1), pred_loss + focal_loss + ssl_loss)


def _tc_post(hl, agc, agn, hlr, agnr, nh, embsel, w1, w2, wb, wo, wob, nw, fs):
    nb = 10
    rb = N // nb
    VP = wo.shape[1]
    cst = lambda i: (0, 0)
    blk = lambda i: (i, 0)
    return pl.pallas_call(
        _tc_post_body,
        grid=(nb,),
        in_specs=[
            pl.BlockSpec((rb, H), blk),
            pl.BlockSpec((rb, H), blk),
            pl.BlockSpec((rb, H), blk),
            pl.BlockSpec((rb, H), blk),
            pl.BlockSpec((rb, H), blk),
            pl.BlockSpec((G, H), cst),
            pl.BlockSpec((G, H), cst),
            pl.BlockSpec((H, H), cst),
            pl.BlockSpec((H, H), cst),
            pl.BlockSpec((1, H), cst),
            pl.BlockSpec((H, VP), cst),
            pl.BlockSpec((1, VP), cst),
            pl.BlockSpec((G, 1), cst),
            pl.BlockSpec((1, 1), cst),
        ],
        out_specs=pl.BlockSpec((1, 1), cst),
        out_shape=jax.ShapeDtypeStruct((1, 1), jnp.float32),
        scratch_shapes=[pltpu.SMEM((2,), jnp.float32)],
    )(hl, agc, agn, hlr, agnr, nh, embsel, w1, w2, wb, wo, wob, nw, fs)


def kernel(ligand_pos, ligand_atom_feature, noise, edge_index, batch_ligand,
           current_atoms, current_atoms_batch, current_wid, next_wid,
           ligand_frontier, W_emb, b_emb, emb_table, W_w, W_b, Wo_w, Wo_b,
           focal_w, focal_b):
    f32 = jnp.float32
    h, fsum = _tc_pre(
        ligand_atom_feature, W_emb, b_emb.reshape(1, H),
        focal_w.reshape(1, H), focal_b.reshape(1, 1),
        ligand_frontier.reshape(N, 1))

    hpk = jnp.concatenate([h[:, :HH], h[:, HH:]], axis=0)
    posn = ligand_pos + 0.3 * noise
    px, py, pz = ligand_pos[:, 0], ligand_pos[:, 1], ligand_pos[:, 2]
    qx, qy, qz = posn[:, 0], posn[:, 1], posn[:, 2]
    src = edge_index[0]
    dst = edge_index[1]
    zeros = jnp.zeros((RPT, HH), f32)

    wc, wn = _sc_weights(px, py, pz, qx, qy, qz, src, dst)
    aggc_pk, aggn_pk, nh_pk, emb_sel = _sc_edges(
        hpk, src, dst, wc, wn,
        current_atoms, current_atoms_batch, current_wid, emb_table, zeros)

    aggc = jnp.concatenate([aggc_pk[:N], aggc_pk[NP:NP + N]], axis=1)
    aggn = jnp.concatenate([aggn_pk[:N], aggn_pk[NP:NP + N]], axis=1)
    nh = jnp.concatenate([nh_pk[:G], nh_pk[G:]], axis=1)
    hlr = jnp.roll(h, -1, axis=0)
    aggnr = jnp.roll(aggn, -1, axis=0)

    VP = 512
    wo_pad = jnp.zeros((H, VP), f32).at[:, :V].set(Wo_w)
    wob_pad = jnp.full((1, VP), -1e30, f32).at[0, :V].set(Wo_b)

    loss = _tc_post(h, aggc, aggn, hlr, aggnr, nh, emb_sel,
                    W_w[:H], W_w[H:], W_b.reshape(1, H), wo_pad, wob_pad,
                    next_wid.reshape(G, 1).astype(jnp.int32), fsum)
    return loss[0, 0]


# speedup vs baseline: 1.0451x; 1.0451x over previous
import functools

import jax
import jax.numpy as jnp
from jax import lax
from jax.experimental import pallas as pl
from jax.experimental.pallas import tpu as pltpu
from jax.experimental.pallas import tpu_sc as plsc

N = 10000
E = 320000
D = 128
H = 128
HH = 64
V = 500
NF = 2048
G = 1024

NC = 2
NS = 16
K = 80
C = 800
NP = 10240
EPT = E // NS
NCH = EPT // K
RPT = NP // NS
FPT = NF // NS
GPW = G // (NC * NS)
NRPT = G // NS


def _tc_pre_body(x_ref, w_ref, b_ref, fw_ref, fb_ref, fr_ref, h_ref, fs_ref):
    i = pl.program_id(0)
    h = jnp.dot(x_ref[...], w_ref[...], preferred_element_type=jnp.float32)
    h = h + b_ref[...]
    h_ref[...] = h
    fp = jnp.sum(h * fw_ref[...], axis=1, keepdims=True) + fb_ref[0, 0]
    t = fr_ref[...]
    bce = jnp.maximum(fp, 0.0) - fp * t + jnp.log1p(jnp.exp(-jnp.abs(fp)))
    part = jnp.sum(bce)

    @pl.when(i == 0)
    def _():
        fs_ref[...] = jnp.zeros((1, 1), jnp.float32)

    fs_ref[...] += jnp.full((1, 1), part)


def _tc_pre(x, w, b, fw, fb, fr):
    nb = 10
    rb = N // nb
    return pl.pallas_call(
        _tc_pre_body,
        grid=(nb,),
        in_specs=[
            pl.BlockSpec((rb, D), lambda i: (i, 0)),
            pl.BlockSpec((D, H), lambda i: (0, 0)),
            pl.BlockSpec((1, H), lambda i: (0, 0)),
            pl.BlockSpec((1, H), lambda i: (0, 0)),
            pl.BlockSpec((1, 1), lambda i: (0, 0)),
            pl.BlockSpec((rb, 1), lambda i: (i, 0)),
        ],
        out_specs=[
            pl.BlockSpec((rb, H), lambda i: (i, 0)),
            pl.BlockSpec((1, 1), lambda i: (0, 0)),
        ],
        out_shape=[
            jax.ShapeDtypeStruct((N, H), jnp.float32),
            jax.ShapeDtypeStruct((1, 1), jnp.float32),
        ],
    )(x, w, b, fw, fb, fr)


WW = 32
WEPT = E // WW
WCH = 2000
WNCH = WEPT // WCH


def _sc_w_body(px, py, pz, qx, qy, qz, srch, dsth, wc_o, wn_o,
               pxv, pyv, pzv, qxv, qyv, qzv, srci, dsti, wcv, wnv):
    c = lax.axis_index("c")
    s = lax.axis_index("s")
    wid = c * NS + s
    pltpu.sync_copy(px, pxv)
    pltpu.sync_copy(py, pyv)
    pltpu.sync_copy(pz, pzv)
    pltpu.sync_copy(qx, qxv)
    pltpu.sync_copy(qy, qyv)
    pltpu.sync_copy(qz, qzv)
    ebase = wid * WEPT

    def chunk_body(i, carry):
        base = ebase + i * WCH
        pltpu.sync_copy(srch.at[pl.ds(base, WCH)], srci)
        pltpu.sync_copy(dsth.at[pl.ds(base, WCH)], dsti)

        def grp(g, cc):
            sl = pl.ds(g * 16, 16)
            s16 = srci[sl]
            d16 = dsti[sl]
            ax = plsc.load_gather(pxv, [s16]) - plsc.load_gather(pxv, [d16])
            ay = plsc.load_gather(pyv, [s16]) - plsc.load_gather(pyv, [d16])
            az = plsc.load_gather(pzv, [s16]) - plsc.load_gather(pzv, [d16])
            wcv[sl] = jnp.exp(-(ax * ax + ay * ay + az * az))
            bx = plsc.load_gather(qxv, [s16]) - plsc.load_gather(qxv, [d16])
            by = plsc.load_gather(qyv, [s16]) - plsc.load_gather(qyv, [d16])
            bz = plsc.load_gather(qzv, [s16]) - plsc.load_gather(qzv, [d16])
            wnv[sl] = jnp.exp(-(bx * bx + by * by + bz * bz))
            return cc

        lax.fori_loop(0, WCH // 16, grp, 0)
        pltpu.sync_copy(wcv, wc_o.at[pl.ds(base, WCH)])
        pltpu.sync_copy(wnv, wn_o.at[pl.ds(base, WCH)])
        return carry

    lax.fori_loop(0, WNCH, chunk_body, 0)


def _sc_weights(px, py, pz, qx, qy, qz, src, dst):
    f32 = jnp.float32
    i32 = jnp.int32
    mesh = plsc.VectorSubcoreMesh(core_axis_name="c", subcore_axis_name="s",
                                  num_cores=NC, num_subcores=NS)
    kfn = pl.kernel(
        _sc_w_body,
        out_type=[
            jax.ShapeDtypeStruct((E,), f32),
            jax.ShapeDtypeStruct((E,), f32),
        ],
        mesh=mesh,
        scratch_types=[
            pltpu.VMEM((N,), f32), pltpu.VMEM((N,), f32), pltpu.VMEM((N,), f32),
            pltpu.VMEM((N,), f32), pltpu.VMEM((N,), f32), pltpu.VMEM((N,), f32),
            pltpu.VMEM((WCH,), i32), pltpu.VMEM((WCH,), i32),
            pltpu.VMEM((WCH,), f32), pltpu.VMEM((WCH,), f32),
        ],
        compiler_params=pltpu.CompilerParams(needs_layout_passes=False,
                                             use_tc_tiling_on_sc=False),
    )
    return kfn(px, py, pz, qx, qy, qz, src, dst)


def _sc_body(hpk, srch, dsth, wch, wnh, cah, cabh, cwh, embh, zeros,
             aggc_o, aggn_o, nhpk_o, embsel_o,
             acc_c, acc_n, nh_acc,
             srci, dsti, dsti2, srcg, wcv, wnv,
             rows0, rows1, rows2, mc0, mc1, mn0, mn1,
             cav, cab2v, cwv, erow,
             gsem0, gsem1, gsem2, scsem0, scsem1, snsem0, snsem1):
    c = lax.axis_index("c")
    s = lax.axis_index("s")
    cN = c * N

    pltpu.sync_copy(zeros, acc_c.at[pl.ds(s * RPT, RPT)])
    pltpu.sync_copy(zeros, acc_n.at[pl.ds(s * RPT, RPT)])
    pltpu.sync_copy(zeros.at[pl.ds(0, NRPT)], nh_acc.at[pl.ds(s * NRPT, NRPT)])
    plsc.subcore_barrier()

    ebase = s * EPT
    NSUB = C // K
    rbufs = (rows0, rows1, rows2)
    mbufs = ((mc0, mn0), (mc1, mn1))
    gsems = (gsem0, gsem1, gsem2)
    ssems = ((scsem0, snsem0), (scsem1, snsem1))

    def chunk_body(i, carry):
        base = ebase + i * C
        pltpu.sync_copy(srch.at[pl.ds(base, C)], srci)
        pltpu.sync_copy(dsth.at[pl.ds(base, C)], dsti)
        pltpu.sync_copy(wch.at[pl.ds(base, C)], wcv)
        pltpu.sync_copy(wnh.at[pl.ds(base, C)], wnv)

        def prep(g, cc):
            sl = pl.ds(g * 16, 16)
            srcg[sl] = srci[sl] + cN
            return cc

        lax.fori_loop(0, C // 16, prep, 0)
        for r in range(NSUB):
            for g in range(K // 16):
                dsti2[r, pl.ds(g * 16, 16)] = dsti[pl.ds(r * K + g * 16, 16)]

        def issue_gather(j):
            return pltpu.async_copy(hpk.at[srcg.at[pl.ds(j * K, K)]],
                                    rbufs[j % 3], gsems[j % 3])

        gds = [issue_gather(0), issue_gather(1)]
        pend = []
        for j in range(NSUB):
            rb = rbufs[j % 3]
            mcb, mnb = mbufs[j % 2]
            if j + 2 < NSUB:
                gds.append(issue_gather(j + 2))
            gds[j].wait()
            if j >= 2:
                so, sn = pend[j - 2]
                so.wait()
                sn.wait()
            woff = j * K

            def scale(g, cc):
                w16c = wcv[pl.ds(woff + g * 16, 16)]
                w16n = wnv[pl.ds(woff + g * 16, 16)]
                for l in range(16):
                    e = g * 16 + l
                    wc_s = w16c[l]
                    wn_s = w16n[l]
                    for q in range(HH // 16):
                        sl = pl.ds(q * 16, 16)
                        r = rb[e, sl]
                        mcb[e, sl] = r * wc_s
                        mnb[e, sl] = r * wn_s
                return cc

            lax.fori_loop(0, K // 16, scale, 0)
            so = pltpu.async_copy(mcb, acc_c.at[dsti2.at[j]],
                                  ssems[j % 2][0], add=True)
            sn = pltpu.async_copy(mnb, acc_n.at[dsti2.at[j]],
                                  ssems[j % 2][1], add=True)
            pend.append((so, sn))
        for so, sn in pend[-2:]:
            so.wait()
            sn.wait()
        return carry

    lax.fori_loop(0, EPT // C, chunk_body, 0)
    plsc.subcore_barrier()

    cNP = c * NP
    pltpu.sync_copy(acc_c.at[pl.ds(s * RPT, RPT)],
                    aggc_o.at[pl.ds(cNP + s * RPT, RPT)])
    pltpu.sync_copy(acc_n.at[pl.ds(s * RPT, RPT)],
                    aggn_o.at[pl.ds(cNP + s * RPT, RPT)])

    pltpu.sync_copy(cah.at[pl.ds(s * FPT, FPT)], cav)
    for g in range(FPT // 16):
        sl = pl.ds(g * 16, 16)
        srcg[sl] = cav[sl] + cN
    pltpu.sync_copy(cabh.at[pl.ds(s * FPT, FPT)], srci.at[pl.ds(0, FPT)])
    for r in range(FPT // 64):
        for g in range(64 // 16):
            cab2v[r, pl.ds(g * 16, 16)] = srci[pl.ds(r * 64 + g * 16, 16)]
    for r in range(FPT // 64):
        ha = rows0.at[pl.ds(0, 64)]
        ga = rows1.at[pl.ds(0, 64)]
        d1 = pltpu.async_copy(hpk.at[srcg.at[pl.ds(r * 64, 64)]], ha, gsem0)
        d2 = pltpu.async_copy(acc_c.at[cav.at[pl.ds(r * 64, 64)]], ga, gsem1)
        d1.wait()
        d2.wait()
        pltpu.sync_copy(ha, nh_acc.at[cab2v.at[r]], add=True)
        pltpu.sync_copy(ga, nh_acc.at[cab2v.at[r]], add=True)
    gbase = c * (G // NC) + s * GPW
    pltpu.sync_copy(cwh.at[pl.ds(gbase, GPW)], cwv)
    for r2 in range(GPW // 16):
        pltpu.async_copy(embh.at[cwv.at[pl.ds(r2 * 16, 16)]],
                         erow, gsem0).wait()
        pltpu.sync_copy(erow, embsel_o.at[pl.ds(gbase + r2 * 16, 16)])
    plsc.subcore_barrier()

    pltpu.sync_copy(nh_acc.at[pl.ds(s * NRPT, NRPT)],
                    nhpk_o.at[pl.ds(c * G + s * NRPT, NRPT)])


def _sc_edges(hpk, src, dst, wc, wn, ca, cab, cw, emb, zeros):
    f32 = jnp.float32
    i32 = jnp.int32
    mesh = plsc.VectorSubcoreMesh(core_axis_name="c", subcore_axis_name="s",
                                  num_cores=NC, num_subcores=NS)
    kfn = pl.kernel(
        _sc_body,
        out_type=[
            jax.ShapeDtypeStruct((2 * NP, HH), f32),
            jax.ShapeDtypeStruct((2 * NP, HH), f32),
            jax.ShapeDtypeStruct((2 * G, HH), f32),
            jax.ShapeDtypeStruct((G, H), f32),
        ],
        mesh=mesh,
        scratch_types=[
            pltpu.VMEM_SHARED((NP, HH), f32),
            pltpu.VMEM_SHARED((NP, HH), f32),
            pltpu.VMEM_SHARED((G, HH), f32),
            pltpu.VMEM((C,), i32), pltpu.VMEM((C,), i32),
            pltpu.VMEM((C // K, K), i32), pltpu.VMEM((C,), i32),
            pltpu.VMEM((C,), f32), pltpu.VMEM((C,), f32),
            pltpu.VMEM((K, HH), f32), pltpu.VMEM((K, HH), f32),
            pltpu.VMEM((K, HH), f32), pltpu.VMEM((K, HH), f32),
            pltpu.VMEM((K, HH), f32), pltpu.VMEM((K, HH), f32),
            pltpu.VMEM((K, HH), f32),
            pltpu.VMEM((FPT,), i32),
            pltpu.VMEM((FPT // 64, 64), i32), pltpu.VMEM((GPW,), i32),
            pltpu.VMEM((16, H), f32),
            pltpu.SemaphoreType.DMA, pltpu.SemaphoreType.DMA,
            pltpu.SemaphoreType.DMA, pltpu.SemaphoreType.DMA,
            pltpu.SemaphoreType.DMA, pltpu.SemaphoreType.DMA,
            pltpu.SemaphoreType.DMA,
        ],
        compiler_params=pltpu.CompilerParams(needs_layout_passes=False,
                                             use_tc_tiling_on_sc=False),
    )
    return kfn(hpk, src, dst, wc, wn, ca, cab, cw, emb, zeros)


def _tc_post_body(hl_ref, agc_ref, agn_ref, hlr_ref, agnr_ref,
                  nh_ref, emb_ref, w1_ref, w2_ref, wb_ref, wo_ref, wob_ref,
                  nw_ref, fs_ref, loss_ref, sacc_ref):
    i = pl.program_id(0)
    nb = pl.num_programs(0)

    @pl.when(i == 0)
    def _():
        sacc_ref[0] = 0.0
        sacc_ref[1] = 0.0

    hl = hl_ref[...]
    h = hl + agc_ref[...]
    h2 = hl + agn_ref[...]
    h2n = hlr_ref[...] + agnr_ref[...]

    def norm(x):
        n = jnp.sqrt(jnp.sum(x * x, axis=1, keepdims=True))
        return x / jnp.maximum(n, 1e-12)

    hn = norm(h)
    hn2 = norm(h2)
    hn2n = norm(h2n)
    pp = jnp.sum(hn * hn2, axis=1, keepdims=True)
    pn = jnp.sum(hn * hn2n, axis=1, keepdims=True)
    sp = jnp.log1p(jnp.exp(-jnp.abs(pp)))
    sacc_ref[0] += jnp.sum(jnp.maximum(pp, 0.0) - pp + sp)
    sn = jnp.log1p(jnp.exp(-jnp.abs(pn)))
    sacc_ref[1] += jnp.sum(jnp.maximum(pn, 0.0) + sn)

    @pl.when(i == nb - 1)
    def _():
        ssl_loss = (sacc_ref[0] / N + sacc_ref[1] / N) * 0.5
        pv = jnp.dot(nh_ref[...], w1_ref[...],
                     preferred_element_type=jnp.float32)
        pv = pv + jnp.dot(emb_ref[...], w2_ref[...],
                          preferred_element_type=jnp.float32)
        pv = jnp.maximum(pv + wb_ref[...], 0.0)
        scores = jnp.dot(pv, wo_ref[...],
                         preferred_element_type=jnp.float32) + wob_ref[...]
        m = jnp.max(scores, axis=1, keepdims=True)
        lz = jnp.log(jnp.sum(jnp.exp(scores - m), axis=1, keepdims=True)) + m
        cols = lax.broadcasted_iota(jnp.int32, scores.shape, 1)
        onehot = cols == nw_ref[...]
        tgt = jnp.sum(jnp.where(onehot, scores, 0.0), axis=1, keepdims=True)
        pred_loss = jnp.sum(lz - tgt) / G
        focal_loss = fs_ref[...][0, 0] / N
        loss_ref[...] = jnp.full((1, 1), pred_loss + focal_loss + ssl_loss)


def _tc_post(hl, agc, agn, hlr, agnr, nh, embsel, w1, w2, wb, wo, wob, nw, fs):
    nb = 10
    rb = N // nb
    VP = wo.shape[1]
    cst = lambda i: (0, 0)
    blk = lambda i: (i, 0)
    return pl.pallas_call(
        _tc_post_body,
        grid=(nb,),
        in_specs=[
            pl.BlockSpec((rb, H), blk),
            pl.BlockSpec((rb, H), blk),
            pl.BlockSpec((rb, H), blk),
            pl.BlockSpec((rb, H), blk),
            pl.BlockSpec((rb, H), blk),
            pl.BlockSpec((G, H), cst),
            pl.BlockSpec((G, H), cst),
            pl.BlockSpec((H, H), cst),
            pl.BlockSpec((H, H), cst),
            pl.BlockSpec((1, H), cst),
            pl.BlockSpec((H, VP), cst),
            pl.BlockSpec((1, VP), cst),
            pl.BlockSpec((G, 1), cst),
            pl.BlockSpec((1, 1), cst),
        ],
        out_specs=pl.BlockSpec((1, 1), cst),
        out_shape=jax.ShapeDtypeStruct((1, 1), jnp.float32),
        scratch_shapes=[pltpu.SMEM((2,), jnp.float32)],
    )(hl, agc, agn, hlr, agnr, nh, embsel, w1, w2, wb, wo, wob, nw, fs)


def kernel(ligand_pos, ligand_atom_feature, noise, edge_index, batch_ligand,
           current_atoms, current_atoms_batch, current_wid, next_wid,
           ligand_frontier, W_emb, b_emb, emb_table, W_w, W_b, Wo_w, Wo_b,
           focal_w, focal_b):
    f32 = jnp.float32
    h, fsum = _tc_pre(
        ligand_atom_feature, W_emb, b_emb.reshape(1, H),
        focal_w.reshape(1, H), focal_b.reshape(1, 1),
        ligand_frontier.reshape(N, 1))

    hpk = jnp.concatenate([h[:, :HH], h[:, HH:]], axis=0)
    posn = ligand_pos + 0.3 * noise
    px, py, pz = ligand_pos[:, 0], ligand_pos[:, 1], ligand_pos[:, 2]
    qx, qy, qz = posn[:, 0], posn[:, 1], posn[:, 2]
    src = edge_index[0]
    dst = edge_index[1]
    zeros = jnp.zeros((RPT, HH), f32)

    wc, wn = _sc_weights(px, py, pz, qx, qy, qz, src, dst)
    aggc_pk, aggn_pk, nh_pk, emb_sel = _sc_edges(
        hpk, src, dst, wc, wn,
        current_atoms, current_atoms_batch, current_wid, emb_table, zeros)

    aggc = jnp.concatenate([aggc_pk[:N], aggc_pk[NP:NP + N]], axis=1)
    aggn = jnp.concatenate([aggn_pk[:N], aggn_pk[NP:NP + N]], axis=1)
    nh = jnp.concatenate([nh_pk[:G], nh_pk[G:]], axis=1)
    hlr = jnp.roll(h, -1, axis=0)
    aggnr = jnp.roll(aggn, -1, axis=0)

    VP = 512
    wo_pad = jnp.zeros((H, VP), f32).at[:, :V].set(Wo_w)
    wob_pad = jnp.full((1, VP), -1e30, f32).at[0, :V].set(Wo_b)

    loss = _tc_post(h, aggc, aggn, hlr, aggnr, nh, emb_sel,
                    W_w[:H], W_w[H:], W_b.reshape(1, H), wo_pad, wob_pad,
                    next_wid.reshape(G, 1).astype(jnp.int32), fsum)
    return loss[0, 0]


# speedup vs baseline: 1.1753x; 1.1246x over previous
import functools

import jax
import jax.numpy as jnp
from jax import lax
from jax.experimental import pallas as pl
from jax.experimental.pallas import tpu as pltpu
from jax.experimental.pallas import tpu_sc as plsc

N = 10000
E = 320000
D = 128
H = 128
HH = 64
V = 500
NF = 2048
G = 1024

NC = 2
NS = 16
K = 80
C = 800
NP = 10240
EPT = E // NS
NCH = EPT // K
RPT = NP // NS
FPT = NF // NS
GPW = G // (NC * NS)
NRPT = G // NS


def _tc_pre_body(x_ref, w_ref, b_ref, fw_ref, fb_ref, fr_ref, h_ref, fs_ref):
    i = pl.program_id(0)
    h = jnp.dot(x_ref[...], w_ref[...], preferred_element_type=jnp.float32)
    h = h + b_ref[...]
    h_ref[...] = h
    fp = jnp.sum(h * fw_ref[...], axis=1, keepdims=True) + fb_ref[0, 0]
    t = fr_ref[...]
    bce = jnp.maximum(fp, 0.0) - fp * t + jnp.log1p(jnp.exp(-jnp.abs(fp)))
    part = jnp.sum(bce)

    @pl.when(i == 0)
    def _():
        fs_ref[...] = jnp.zeros((1, 1), jnp.float32)

    fs_ref[...] += jnp.full((1, 1), part)


def _tc_pre(x, w, b, fw, fb, fr):
    nb = 10
    rb = N // nb
    return pl.pallas_call(
        _tc_pre_body,
        grid=(nb,),
        in_specs=[
            pl.BlockSpec((rb, D), lambda i: (i, 0)),
            pl.BlockSpec((D, H), lambda i: (0, 0)),
            pl.BlockSpec((1, H), lambda i: (0, 0)),
            pl.BlockSpec((1, H), lambda i: (0, 0)),
            pl.BlockSpec((1, 1), lambda i: (0, 0)),
            pl.BlockSpec((rb, 1), lambda i: (i, 0)),
        ],
        out_specs=[
            pl.BlockSpec((rb, H), lambda i: (i, 0)),
            pl.BlockSpec((1, 1), lambda i: (0, 0)),
        ],
        out_shape=[
            jax.ShapeDtypeStruct((N, H), jnp.float32),
            jax.ShapeDtypeStruct((1, 1), jnp.float32),
        ],
    )(x, w, b, fw, fb, fr)


WW = 32
WEPT = E // WW
WCH = 2000
WNCH = WEPT // WCH


def _sc_w_body(px, py, pz, qx, qy, qz, srch, dsth, wc_o, wn_o,
               pxv, pyv, pzv, qxv, qyv, qzv, srci, dsti, wcv, wnv):
    c = lax.axis_index("c")
    s = lax.axis_index("s")
    wid = c * NS + s
    pltpu.sync_copy(px, pxv)
    pltpu.sync_copy(py, pyv)
    pltpu.sync_copy(pz, pzv)
    pltpu.sync_copy(qx, qxv)
    pltpu.sync_copy(qy, qyv)
    pltpu.sync_copy(qz, qzv)
    ebase = wid * WEPT

    def chunk_body(i, carry):
        base = ebase + i * WCH
        pltpu.sync_copy(srch.at[pl.ds(base, WCH)], srci)
        pltpu.sync_copy(dsth.at[pl.ds(base, WCH)], dsti)

        def grp(g, cc):
            sl = pl.ds(g * 16, 16)
            s16 = srci[sl]
            d16 = dsti[sl]
            ax = plsc.load_gather(pxv, [s16]) - plsc.load_gather(pxv, [d16])
            ay = plsc.load_gather(pyv, [s16]) - plsc.load_gather(pyv, [d16])
            az = plsc.load_gather(pzv, [s16]) - plsc.load_gather(pzv, [d16])
            wcv[sl] = jnp.exp(-(ax * ax + ay * ay + az * az))
            bx = plsc.load_gather(qxv, [s16]) - plsc.load_gather(qxv, [d16])
            by = plsc.load_gather(qyv, [s16]) - plsc.load_gather(qyv, [d16])
            bz = plsc.load_gather(qzv, [s16]) - plsc.load_gather(qzv, [d16])
            wnv[sl] = jnp.exp(-(bx * bx + by * by + bz * bz))
            return cc

        lax.fori_loop(0, WCH // 16, grp, 0)
        pltpu.sync_copy(wcv, wc_o.at[pl.ds(base, WCH)])
        pltpu.sync_copy(wnv, wn_o.at[pl.ds(base, WCH)])
        return carry

    lax.fori_loop(0, WNCH, chunk_body, 0)


def _sc_weights(px, py, pz, qx, qy, qz, src, dst):
    f32 = jnp.float32
    i32 = jnp.int32
    mesh = plsc.VectorSubcoreMesh(core_axis_name="c", subcore_axis_name="s",
                                  num_cores=NC, num_subcores=NS)
    kfn = pl.kernel(
        _sc_w_body,
        out_type=[
            jax.ShapeDtypeStruct((E,), f32),
            jax.ShapeDtypeStruct((E,), f32),
        ],
        mesh=mesh,
        scratch_types=[
            pltpu.VMEM((N,), f32), pltpu.VMEM((N,), f32), pltpu.VMEM((N,), f32),
            pltpu.VMEM((N,), f32), pltpu.VMEM((N,), f32), pltpu.VMEM((N,), f32),
            pltpu.VMEM((WCH,), i32), pltpu.VMEM((WCH,), i32),
            pltpu.VMEM((WCH,), f32), pltpu.VMEM((WCH,), f32),
        ],
        compiler_params=pltpu.CompilerParams(needs_layout_passes=False,
                                             use_tc_tiling_on_sc=False),
    )
    return kfn(px, py, pz, qx, qy, qz, src, dst)


def _sc_body(hpk, sdwh, cah, cabh, cwh, embh, zeros,
             aggc_o, aggn_o, nhpk_o, embsel_o,
             acc_c, acc_n, nh_acc,
             stgv, dsti2, srcg,
             rows0, rows1, rows2, mc0, mc1, mn0, mn1,
             cav, cab2v, cwv, erow,
             gsem0, gsem1, gsem2, scsem0, scsem1, snsem0, snsem1):
    c = lax.axis_index("c")
    s = lax.axis_index("s")
    cN = c * N

    pltpu.sync_copy(zeros, acc_c.at[pl.ds(s * RPT, RPT)])
    pltpu.sync_copy(zeros, acc_n.at[pl.ds(s * RPT, RPT)])
    pltpu.sync_copy(zeros.at[pl.ds(0, NRPT)], nh_acc.at[pl.ds(s * NRPT, NRPT)])
    plsc.subcore_barrier()

    ebase = s * EPT
    NSUB = C // K
    rbufs = (rows0, rows1, rows2)
    mbufs = ((mc0, mn0), (mc1, mn1))
    gsems = (gsem0, gsem1, gsem2)
    ssems = ((scsem0, snsem0), (scsem1, snsem1))

    def chunk_body(i, carry):
        row = s * (EPT // C) + i
        pltpu.sync_copy(sdwh.at[row], stgv)

        def prep(g, cc):
            sl = pl.ds(g * 16, 16)
            srcg[sl] = stgv[0, sl] + cN
            return cc

        lax.fori_loop(0, C // 16, prep, 0)
        for r in range(NSUB):
            for g in range(K // 16):
                dsti2[r, pl.ds(g * 16, 16)] = stgv[1, pl.ds(r * K + g * 16, 16)]

        def issue_gather(j):
            return pltpu.async_copy(hpk.at[srcg.at[pl.ds(j * K, K)]],
                                    rbufs[j % 3], gsems[j % 3])

        gds = [issue_gather(0), issue_gather(1)]
        pend = []
        for j in range(NSUB):
            rb = rbufs[j % 3]
            mcb, mnb = mbufs[j % 2]
            if j + 2 < NSUB:
                gds.append(issue_gather(j + 2))
            gds[j].wait()
            if j >= 2:
                so, sn = pend[j - 2]
                so.wait()
                sn.wait()
            woff = j * K

            def scale(g, cc):
                w16c = plsc.bitcast(stgv[2, pl.ds(woff + g * 16, 16)],
                                    jnp.float32)
                w16n = plsc.bitcast(stgv[3, pl.ds(woff + g * 16, 16)],
                                    jnp.float32)
                for l in range(16):
                    e = g * 16 + l
                    wc_s = w16c[l]
                    wn_s = w16n[l]
                    for q in range(HH // 16):
                        sl = pl.ds(q * 16, 16)
                        r = rb[e, sl]
                        mcb[e, sl] = r * wc_s
                        mnb[e, sl] = r * wn_s
                return cc

            lax.fori_loop(0, K // 16, scale, 0)
            so = pltpu.async_copy(mcb, acc_c.at[dsti2.at[j]],
                                  ssems[j % 2][0], add=True)
            sn = pltpu.async_copy(mnb, acc_n.at[dsti2.at[j]],
                                  ssems[j % 2][1], add=True)
            pend.append((so, sn))
        for so, sn in pend[-2:]:
            so.wait()
            sn.wait()
        return carry

    lax.fori_loop(0, EPT // C, chunk_body, 0)
    plsc.subcore_barrier()

    cNP = c * NP
    pltpu.sync_copy(acc_c.at[pl.ds(s * RPT, RPT)],
                    aggc_o.at[pl.ds(cNP + s * RPT, RPT)])
    pltpu.sync_copy(acc_n.at[pl.ds(s * RPT, RPT)],
                    aggn_o.at[pl.ds(cNP + s * RPT, RPT)])

    pltpu.sync_copy(cah.at[pl.ds(s * FPT, FPT)], cav)
    for g in range(FPT // 16):
        sl = pl.ds(g * 16, 16)
        srcg[sl] = cav[sl] + cN
    pltpu.sync_copy(cabh.at[pl.ds(s * FPT, FPT)], stgv.at[0, pl.ds(0, FPT)])
    for r in range(FPT // 64):
        for g in range(64 // 16):
            cab2v[r, pl.ds(g * 16, 16)] = stgv[0, pl.ds(r * 64 + g * 16, 16)]
    for r in range(FPT // 64):
        ha = rows0.at[pl.ds(0, 64)]
        ga = rows1.at[pl.ds(0, 64)]
        d1 = pltpu.async_copy(hpk.at[srcg.at[pl.ds(r * 64, 64)]], ha, gsem0)
        d2 = pltpu.async_copy(acc_c.at[cav.at[pl.ds(r * 64, 64)]], ga, gsem1)
        d1.wait()
        d2.wait()
        pltpu.sync_copy(ha, nh_acc.at[cab2v.at[r]], add=True)
        pltpu.sync_copy(ga, nh_acc.at[cab2v.at[r]], add=True)
    gbase = c * (G // NC) + s * GPW
    pltpu.sync_copy(cwh.at[pl.ds(gbase, GPW)], cwv)
    for r2 in range(GPW // 16):
        pltpu.async_copy(embh.at[cwv.at[pl.ds(r2 * 16, 16)]],
                         erow, gsem0).wait()
        pltpu.sync_copy(erow, embsel_o.at[pl.ds(gbase + r2 * 16, 16)])
    plsc.subcore_barrier()

    pltpu.sync_copy(nh_acc.at[pl.ds(s * NRPT, NRPT)],
                    nhpk_o.at[pl.ds(c * G + s * NRPT, NRPT)])


def _sc_edges(hpk, sdw, ca, cab, cw, emb, zeros):
    f32 = jnp.float32
    i32 = jnp.int32
    mesh = plsc.VectorSubcoreMesh(core_axis_name="c", subcore_axis_name="s",
                                  num_cores=NC, num_subcores=NS)
    kfn = pl.kernel(
        _sc_body,
        out_type=[
            jax.ShapeDtypeStruct((2 * NP, HH), f32),
            jax.ShapeDtypeStruct((2 * NP, HH), f32),
            jax.ShapeDtypeStruct((2 * G, HH), f32),
            jax.ShapeDtypeStruct((G, H), f32),
        ],
        mesh=mesh,
        scratch_types=[
            pltpu.VMEM_SHARED((NP, HH), f32),
            pltpu.VMEM_SHARED((NP, HH), f32),
            pltpu.VMEM_SHARED((G, HH), f32),
            pltpu.VMEM((4, C), i32),
            pltpu.VMEM((C // K, K), i32), pltpu.VMEM((C,), i32),
            pltpu.VMEM((K, HH), f32), pltpu.VMEM((K, HH), f32),
            pltpu.VMEM((K, HH), f32), pltpu.VMEM((K, HH), f32),
            pltpu.VMEM((K, HH), f32), pltpu.VMEM((K, HH), f32),
            pltpu.VMEM((K, HH), f32),
            pltpu.VMEM((FPT,), i32),
            pltpu.VMEM((FPT // 64, 64), i32), pltpu.VMEM((GPW,), i32),
            pltpu.VMEM((16, H), f32),
            pltpu.SemaphoreType.DMA, pltpu.SemaphoreType.DMA,
            pltpu.SemaphoreType.DMA, pltpu.SemaphoreType.DMA,
            pltpu.SemaphoreType.DMA, pltpu.SemaphoreType.DMA,
            pltpu.SemaphoreType.DMA,
        ],
        compiler_params=pltpu.CompilerParams(needs_layout_passes=False,
                                             use_tc_tiling_on_sc=False),
    )
    return kfn(hpk, sdw, ca, cab, cw, emb, zeros)


def _tc_post_body(hl_ref, agc_ref, agn_ref, hlr_ref, agnr_ref,
                  nh_ref, emb_ref, w1_ref, w2_ref, wb_ref, wo_ref, wob_ref,
                  nw_ref, fs_ref, loss_ref, sacc_ref):
    i = pl.program_id(0)
    nb = pl.num_programs(0)

    @pl.when(i == 0)
    def _():
        sacc_ref[0] = 0.0
        sacc_ref[1] = 0.0

    hl = hl_ref[...]
    h = hl + agc_ref[...]
    h2 = hl + agn_ref[...]
    h2n = hlr_ref[...] + agnr_ref[...]

    def norm(x):
        n = jnp.sqrt(jnp.sum(x * x, axis=1, keepdims=True))
        return x / jnp.maximum(n, 1e-12)

    hn = norm(h)
    hn2 = norm(h2)
    hn2n = norm(h2n)
    pp = jnp.sum(hn * hn2, axis=1, keepdims=True)
    pn = jnp.sum(hn * hn2n, axis=1, keepdims=True)
    sp = jnp.log1p(jnp.exp(-jnp.abs(pp)))
    sacc_ref[0] += jnp.sum(jnp.maximum(pp, 0.0) - pp + sp)
    sn = jnp.log1p(jnp.exp(-jnp.abs(pn)))
    sacc_ref[1] += jnp.sum(jnp.maximum(pn, 0.0) + sn)

    @pl.when(i == nb - 1)
    def _():
        ssl_loss = (sacc_ref[0] / N + sacc_ref[1] / N) * 0.5
        pv = jnp.dot(nh_ref[...], w1_ref[...],
                     preferred_element_type=jnp.float32)
        pv = pv + jnp.dot(emb_ref[...], w2_ref[...],
                          preferred_element_type=jnp.float32)
        pv = jnp.maximum(pv + wb_ref[...], 0.0)
        scores = jnp.dot(pv, wo_ref[...],
                         preferred_element_type=jnp.float32) + wob_ref[...]
        m = jnp.max(scores, axis=1, keepdims=True)
        lz = jnp.log(jnp.sum(jnp.exp(scores - m), axis=1, keepdims=True)) + m
        cols = lax.broadcasted_iota(jnp.int32, scores.shape, 1)
        onehot = cols == nw_ref[...]
        tgt = jnp.sum(jnp.where(onehot, scores, 0.0), axis=1, keepdims=True)
        pred_loss = jnp.sum(lz - tgt) / G
        focal_loss = fs_ref[...][0, 0] / N
        loss_ref[...] = jnp.full((1, 1), pred_loss + focal_loss + ssl_loss)


def _tc_post(hl, agc, agn, hlr, agnr, nh, embsel, w1, w2, wb, wo, wob, nw, fs):
    nb = 10
    rb = N // nb
    VP = wo.shape[1]
    cst = lambda i: (0, 0)
    blk = lambda i: (i, 0)
    return pl.pallas_call(
        _tc_post_body,
        grid=(nb,),
        in_specs=[
            pl.BlockSpec((rb, H), blk),
            pl.BlockSpec((rb, H), blk),
            pl.BlockSpec((rb, H), blk),
            pl.BlockSpec((rb, H), blk),
            pl.BlockSpec((rb, H), blk),
            pl.BlockSpec((G, H), cst),
            pl.BlockSpec((G, H), cst),
            pl.BlockSpec((H, H), cst),
            pl.BlockSpec((H, H), cst),
            pl.BlockSpec((1, H), cst),
            pl.BlockSpec((H, VP), cst),
            pl.BlockSpec((1, VP), cst),
            pl.BlockSpec((G, 1), cst),
            pl.BlockSpec((1, 1), cst),
        ],
        out_specs=pl.BlockSpec((1, 1), cst),
        out_shape=jax.ShapeDtypeStruct((1, 1), jnp.float32),
        scratch_shapes=[pltpu.SMEM((2,), jnp.float32)],
    )(hl, agc, agn, hlr, agnr, nh, embsel, w1, w2, wb, wo, wob, nw, fs)


def kernel(ligand_pos, ligand_atom_feature, noise, edge_index, batch_ligand,
           current_atoms, current_atoms_batch, current_wid, next_wid,
           ligand_frontier, W_emb, b_emb, emb_table, W_w, W_b, Wo_w, Wo_b,
           focal_w, focal_b):
    f32 = jnp.float32
    h, fsum = _tc_pre(
        ligand_atom_feature, W_emb, b_emb.reshape(1, H),
        focal_w.reshape(1, H), focal_b.reshape(1, 1),
        ligand_frontier.reshape(N, 1))

    hpk = jnp.concatenate([h[:, :HH], h[:, HH:]], axis=0)
    posn = ligand_pos + 0.3 * noise
    px, py, pz = ligand_pos[:, 0], ligand_pos[:, 1], ligand_pos[:, 2]
    qx, qy, qz = posn[:, 0], posn[:, 1], posn[:, 2]
    src = edge_index[0]
    dst = edge_index[1]
    zeros = jnp.zeros((RPT, HH), f32)

    wc, wn = _sc_weights(px, py, pz, qx, qy, qz, src, dst)
    sdw = jnp.stack([
        src.reshape(E // C, C),
        dst.reshape(E // C, C),
        jax.lax.bitcast_convert_type(wc, jnp.int32).reshape(E // C, C),
        jax.lax.bitcast_convert_type(wn, jnp.int32).reshape(E // C, C),
    ], axis=1)
    aggc_pk, aggn_pk, nh_pk, emb_sel = _sc_edges(
        hpk, sdw,
        current_atoms, current_atoms_batch, current_wid, emb_table, zeros)

    aggc = jnp.concatenate([aggc_pk[:N], aggc_pk[NP:NP + N]], axis=1)
    aggn = jnp.concatenate([aggn_pk[:N], aggn_pk[NP:NP + N]], axis=1)
    nh = jnp.concatenate([nh_pk[:G], nh_pk[G:]], axis=1)
    hlr = jnp.roll(h, -1, axis=0)
    aggnr = jnp.roll(aggn, -1, axis=0)

    VP = 512
    wo_pad = jnp.zeros((H, VP), f32).at[:, :V].set(Wo_w)
    wob_pad = jnp.full((1, VP), -1e30, f32).at[0, :V].set(Wo_b)

    loss = _tc_post(h, aggc, aggn, hlr, aggnr, nh, emb_sel,
                    W_w[:H], W_w[H:], W_b.reshape(1, H), wo_pad, wob_pad,
                    next_wid.reshape(G, 1).astype(jnp.int32), fsum)
    return loss[0, 0]


# speedup vs baseline: 1.2887x; 1.0965x over previous
import functools

import jax
import jax.numpy as jnp
from jax import lax
from jax.experimental import pallas as pl
from jax.experimental.pallas import tpu as pltpu
from jax.experimental.pallas import tpu_sc as plsc

N = 10000
E = 320000
D = 128
H = 128
HH = 64
V = 500
NF = 2048
G = 1024

NC = 2
NS = 16
K = 80
C = 800
NP = 10000
EPT = E // NS
NCH = EPT // K
RPT = NP // NS
FPT = NF // NS
GPW = G // (NC * NS)
NRPT = G // NS


def _tc_pre_body(x_ref, w_ref, b_ref, fw_ref, fb_ref, fr_ref,
                 hpk_ref, fs_ref, fp_scr):
    i = pl.program_id(0)
    j = pl.program_id(1)
    h = jnp.dot(x_ref[...], w_ref[...][0], preferred_element_type=jnp.float32)
    h = h + b_ref[...][0]
    hpk_ref[...] = h
    fp_j = jnp.sum(h * fw_ref[...][0], axis=1, keepdims=True)

    @pl.when(j == 0)
    def _():
        fp_scr[...] = fp_j

    @pl.when(j == 1)
    def _():
        fp = fp_scr[...] + fp_j + fb_ref[0, 0]
        t = fr_ref[...]
        bce = jnp.maximum(fp, 0.0) - fp * t + jnp.log1p(jnp.exp(-jnp.abs(fp)))
        part = jnp.full((1, 1), jnp.sum(bce))

        @pl.when(i == 0)
        def _():
            fs_ref[...] = jnp.zeros((1, 1), jnp.float32)

        fs_ref[...] += part


def _tc_pre(x, w, b, fw, fb, fr):
    nb = 10
    rb = N // nb
    return pl.pallas_call(
        _tc_pre_body,
        grid=(nb, 2),
        in_specs=[
            pl.BlockSpec((rb, D), lambda i, j: (i, 0)),
            pl.BlockSpec((1, D, HH), lambda i, j: (j, 0, 0)),
            pl.BlockSpec((1, 1, HH), lambda i, j: (j, 0, 0)),
            pl.BlockSpec((1, 1, HH), lambda i, j: (j, 0, 0)),
            pl.BlockSpec((1, 1), lambda i, j: (0, 0)),
            pl.BlockSpec((rb, 1), lambda i, j: (i, 0)),
        ],
        out_specs=[
            pl.BlockSpec((rb, HH), lambda i, j: (j * nb + i, 0)),
            pl.BlockSpec((1, 1), lambda i, j: (0, 0)),
        ],
        out_shape=[
            jax.ShapeDtypeStruct((2 * N, HH), jnp.float32),
            jax.ShapeDtypeStruct((1, 1), jnp.float32),
        ],
        scratch_shapes=[pltpu.VMEM((rb, 1), jnp.float32)],
    )(x, w, b, fw, fb, fr)


WW = 32
WEPT = E // WW
WCH = 2000
WNCH = WEPT // WCH


def _sc_w_body(px, py, pz, qx, qy, qz, srch, dsth, wc_o, wn_o,
               pxv, pyv, pzv, qxv, qyv, qzv, srci, dsti, wcv, wnv):
    c = lax.axis_index("c")
    s = lax.axis_index("s")
    wid = c * NS + s
    pltpu.sync_copy(px, pxv)
    pltpu.sync_copy(py, pyv)
    pltpu.sync_copy(pz, pzv)
    pltpu.sync_copy(qx, qxv)
    pltpu.sync_copy(qy, qyv)
    pltpu.sync_copy(qz, qzv)
    ebase = wid * WEPT

    def chunk_body(i, carry):
        base = ebase + i * WCH
        pltpu.sync_copy(srch.at[pl.ds(base, WCH)], srci)
        pltpu.sync_copy(dsth.at[pl.ds(base, WCH)], dsti)

        def grp(g, cc):
            sl = pl.ds(g * 16, 16)
            s16 = srci[sl]
            d16 = dsti[sl]
            ax = plsc.load_gather(pxv, [s16]) - plsc.load_gather(pxv, [d16])
            ay = plsc.load_gather(pyv, [s16]) - plsc.load_gather(pyv, [d16])
            az = plsc.load_gather(pzv, [s16]) - plsc.load_gather(pzv, [d16])
            wcv[sl] = jnp.exp(-(ax * ax + ay * ay + az * az))
            bx = plsc.load_gather(qxv, [s16]) - plsc.load_gather(qxv, [d16])
            by = plsc.load_gather(qyv, [s16]) - plsc.load_gather(qyv, [d16])
            bz = plsc.load_gather(qzv, [s16]) - plsc.load_gather(qzv, [d16])
            wnv[sl] = jnp.exp(-(bx * bx + by * by + bz * bz))
            return cc

        lax.fori_loop(0, WCH // 16, grp, 0)
        pltpu.sync_copy(wcv, wc_o.at[pl.ds(base, WCH)])
        pltpu.sync_copy(wnv, wn_o.at[pl.ds(base, WCH)])
        return carry

    lax.fori_loop(0, WNCH, chunk_body, 0)


def _sc_weights(px, py, pz, qx, qy, qz, src, dst):
    f32 = jnp.float32
    i32 = jnp.int32
    mesh = plsc.VectorSubcoreMesh(core_axis_name="c", subcore_axis_name="s",
                                  num_cores=NC, num_subcores=NS)
    kfn = pl.kernel(
        _sc_w_body,
        out_type=[
            jax.ShapeDtypeStruct((E,), f32),
            jax.ShapeDtypeStruct((E,), f32),
        ],
        mesh=mesh,
        scratch_types=[
            pltpu.VMEM((N,), f32), pltpu.VMEM((N,), f32), pltpu.VMEM((N,), f32),
            pltpu.VMEM((N,), f32), pltpu.VMEM((N,), f32), pltpu.VMEM((N,), f32),
            pltpu.VMEM((WCH,), i32), pltpu.VMEM((WCH,), i32),
            pltpu.VMEM((WCH,), f32), pltpu.VMEM((WCH,), f32),
        ],
        compiler_params=pltpu.CompilerParams(needs_layout_passes=False,
                                             use_tc_tiling_on_sc=False),
    )
    return kfn(px, py, pz, qx, qy, qz, src, dst)


def _sc_body(hpk, sdwh, cah, cabh, cwh, embh, zeros,
             aggc_o, aggn_o, nhpk_o, embsel_o,
             acc_c, acc_n, nh_acc,
             stgv, dsti2, srcg,
             rows0, rows1, rows2, mc0, mc1, mn0, mn1,
             cav, cab2v, cwv, erow,
             gsem0, gsem1, gsem2, scsem0, scsem1, snsem0, snsem1):
    c = lax.axis_index("c")
    s = lax.axis_index("s")
    cN = c * N

    pltpu.sync_copy(zeros, acc_c.at[pl.ds(s * RPT, RPT)])
    pltpu.sync_copy(zeros, acc_n.at[pl.ds(s * RPT, RPT)])
    pltpu.sync_copy(zeros.at[pl.ds(0, NRPT)], nh_acc.at[pl.ds(s * NRPT, NRPT)])
    plsc.subcore_barrier()

    ebase = s * EPT
    NSUB = C // K
    rbufs = (rows0, rows1, rows2)
    mbufs = ((mc0, mn0), (mc1, mn1))
    gsems = (gsem0, gsem1, gsem2)
    ssems = ((scsem0, snsem0), (scsem1, snsem1))

    def chunk_body(i, carry):
        row = s * (EPT // C) + i
        pltpu.sync_copy(sdwh.at[row], stgv)

        def prep(g, cc):
            sl = pl.ds(g * 16, 16)
            srcg[sl] = stgv[0, sl] + cN
            return cc

        lax.fori_loop(0, C // 16, prep, 0)
        for r in range(NSUB):
            for g in range(K // 16):
                dsti2[r, pl.ds(g * 16, 16)] = stgv[1, pl.ds(r * K + g * 16, 16)]

        def issue_gather(j):
            return pltpu.async_copy(hpk.at[srcg.at[pl.ds(j * K, K)]],
                                    rbufs[j % 3], gsems[j % 3])

        gds = [issue_gather(0), issue_gather(1)]
        pend = []
        for j in range(NSUB):
            rb = rbufs[j % 3]
            mcb, mnb = mbufs[j % 2]
            if j + 2 < NSUB:
                gds.append(issue_gather(j + 2))
            gds[j].wait()
            if j >= 2:
                so, sn = pend[j - 2]
                so.wait()
                sn.wait()
            woff = j * K

            def scale(g, cc):
                w16c = plsc.bitcast(stgv[2, pl.ds(woff + g * 16, 16)],
                                    jnp.float32)
                w16n = plsc.bitcast(stgv[3, pl.ds(woff + g * 16, 16)],
                                    jnp.float32)
                for l in range(16):
                    e = g * 16 + l
                    wc_s = w16c[l]
                    wn_s = w16n[l]
                    for q in range(HH // 16):
                        sl = pl.ds(q * 16, 16)
                        r = rb[e, sl]
                        mcb[e, sl] = r * wc_s
                        mnb[e, sl] = r * wn_s
                return cc

            lax.fori_loop(0, K // 16, scale, 0)
            so = pltpu.async_copy(mcb, acc_c.at[dsti2.at[j]],
                                  ssems[j % 2][0], add=True)
            sn = pltpu.async_copy(mnb, acc_n.at[dsti2.at[j]],
                                  ssems[j % 2][1], add=True)
            pend.append((so, sn))
        for so, sn in pend[-2:]:
            so.wait()
            sn.wait()
        return carry

    lax.fori_loop(0, EPT // C, chunk_body, 0)
    plsc.subcore_barrier()

    cNP = c * NP
    pltpu.sync_copy(acc_c.at[pl.ds(s * RPT, RPT)],
                    aggc_o.at[pl.ds(cNP + s * RPT, RPT)])
    pltpu.sync_copy(acc_n.at[pl.ds(s * RPT, RPT)],
                    aggn_o.at[pl.ds(cNP + s * RPT, RPT)])

    pltpu.sync_copy(cah.at[pl.ds(s * FPT, FPT)], cav)
    for g in range(FPT // 16):
        sl = pl.ds(g * 16, 16)
        srcg[sl] = cav[sl] + cN
    pltpu.sync_copy(cabh.at[pl.ds(s * FPT, FPT)], stgv.at[0, pl.ds(0, FPT)])
    for r in range(FPT // 64):
        for g in range(64 // 16):
            cab2v[r, pl.ds(g * 16, 16)] = stgv[0, pl.ds(r * 64 + g * 16, 16)]
    for r in range(FPT // 64):
        ha = rows0.at[pl.ds(0, 64)]
        ga = rows1.at[pl.ds(0, 64)]
        d1 = pltpu.async_copy(hpk.at[srcg.at[pl.ds(r * 64, 64)]], ha, gsem0)
        d2 = pltpu.async_copy(acc_c.at[cav.at[pl.ds(r * 64, 64)]], ga, gsem1)
        d1.wait()
        d2.wait()
        pltpu.sync_copy(ha, nh_acc.at[cab2v.at[r]], add=True)
        pltpu.sync_copy(ga, nh_acc.at[cab2v.at[r]], add=True)
    gbase = c * (G // NC) + s * GPW
    pltpu.sync_copy(cwh.at[pl.ds(gbase, GPW)], cwv)
    for r2 in range(GPW // 16):
        pltpu.async_copy(embh.at[cwv.at[pl.ds(r2 * 16, 16)]],
                         erow, gsem0).wait()
        pltpu.sync_copy(erow, embsel_o.at[pl.ds(gbase + r2 * 16, 16)])
    plsc.subcore_barrier()

    pltpu.sync_copy(nh_acc.at[pl.ds(s * NRPT, NRPT)],
                    nhpk_o.at[pl.ds(c * G + s * NRPT, NRPT)])


def _sc_edges(hpk, sdw, ca, cab, cw, emb, zeros):
    f32 = jnp.float32
    i32 = jnp.int32
    mesh = plsc.VectorSubcoreMesh(core_axis_name="c", subcore_axis_name="s",
                                  num_cores=NC, num_subcores=NS)
    kfn = pl.kernel(
        _sc_body,
        out_type=[
            jax.ShapeDtypeStruct((2 * NP, HH), f32),
            jax.ShapeDtypeStruct((2 * NP, HH), f32),
            jax.ShapeDtypeStruct((2 * G, HH), f32),
            jax.ShapeDtypeStruct((G, H), f32),
        ],
        mesh=mesh,
        scratch_types=[
            pltpu.VMEM_SHARED((NP, HH), f32),
            pltpu.VMEM_SHARED((NP, HH), f32),
            pltpu.VMEM_SHARED((G, HH), f32),
            pltpu.VMEM((4, C), i32),
            pltpu.VMEM((C // K, K), i32), pltpu.VMEM((C,), i32),
            pltpu.VMEM((K, HH), f32), pltpu.VMEM((K, HH), f32),
            pltpu.VMEM((K, HH), f32), pltpu.VMEM((K, HH), f32),
            pltpu.VMEM((K, HH), f32), pltpu.VMEM((K, HH), f32),
            pltpu.VMEM((K, HH), f32),
            pltpu.VMEM((FPT,), i32),
            pltpu.VMEM((FPT // 64, 64), i32), pltpu.VMEM((GPW,), i32),
            pltpu.VMEM((16, H), f32),
            pltpu.SemaphoreType.DMA, pltpu.SemaphoreType.DMA,
            pltpu.SemaphoreType.DMA, pltpu.SemaphoreType.DMA,
            pltpu.SemaphoreType.DMA, pltpu.SemaphoreType.DMA,
            pltpu.SemaphoreType.DMA,
        ],
        compiler_params=pltpu.CompilerParams(needs_layout_passes=False,
                                             use_tc_tiling_on_sc=False),
    )
    return kfn(hpk, sdw, ca, cab, cw, emb, zeros)


def _bce1(x):
    return jnp.maximum(x, 0.0) - x + jnp.log1p(jnp.exp(-jnp.abs(x)))


def _bce0(x):
    return jnp.maximum(x, 0.0) + jnp.log1p(jnp.exp(-jnp.abs(x)))


def _tc_post_body(hlo_ref, hhi_ref, aclo_ref, achi_ref, anlo_ref, anhi_ref,
                  nhlo_ref, nhhi_ref, emb_ref, w1_ref, w2_ref, wb_ref,
                  wo_ref, wob_ref, nw_ref, fs_ref, loss_ref,
                  sacc_ref, prev_ref, first_ref):
    i = pl.program_id(0)
    nb = pl.num_programs(0)

    @pl.when(i == 0)
    def _():
        sacc_ref[0] = 0.0
        sacc_ref[1] = 0.0

    hl = jnp.concatenate([hlo_ref[...], hhi_ref[...]], axis=1)
    h = hl + jnp.concatenate([aclo_ref[...], achi_ref[...]], axis=1)
    h2 = hl + jnp.concatenate([anlo_ref[...], anhi_ref[...]], axis=1)

    def norm(x):
        n = jnp.sqrt(jnp.sum(x * x, axis=1, keepdims=True))
        return x / jnp.maximum(n, 1e-12)

    hn = norm(h)
    hn2 = norm(h2)
    pp = jnp.sum(hn * hn2, axis=1, keepdims=True)
    sacc_ref[0] += jnp.sum(_bce1(pp))
    rb = hn.shape[0]
    pn_local = jnp.sum(hn[:rb - 1] * hn2[1:], axis=1, keepdims=True)
    sacc_ref[1] += jnp.sum(_bce0(pn_local))

    @pl.when(i > 0)
    def _():
        cross = jnp.sum(prev_ref[...] * hn2[0:1], axis=1, keepdims=True)
        sacc_ref[1] += jnp.sum(_bce0(cross))

    @pl.when(i == 0)
    def _():
        first_ref[...] = hn2[0:1]

    prev_ref[...] = hn[rb - 1:rb]

    @pl.when(i == nb - 1)
    def _():
        wrap = jnp.sum(hn[rb - 1:rb] * first_ref[...], axis=1, keepdims=True)
        sacc_ref[1] += jnp.sum(_bce0(wrap))
        ssl_loss = (sacc_ref[0] / N + sacc_ref[1] / N) * 0.5
        nh = jnp.concatenate([nhlo_ref[...], nhhi_ref[...]], axis=1)
        pv = jnp.dot(nh, w1_ref[...],
                     preferred_element_type=jnp.float32)
        pv = pv + jnp.dot(emb_ref[...], w2_ref[...],
                          preferred_element_type=jnp.float32)
        pv = jnp.maximum(pv + wb_ref[...], 0.0)
        scores = jnp.dot(pv, wo_ref[...],
                         preferred_element_type=jnp.float32) + wob_ref[...]
        m = jnp.max(scores, axis=1, keepdims=True)
        lz = jnp.log(jnp.sum(jnp.exp(scores - m), axis=1, keepdims=True)) + m
        cols = lax.broadcasted_iota(jnp.int32, scores.shape, 1)
        onehot = cols == nw_ref[...]
        tgt = jnp.sum(jnp.where(onehot, scores, 0.0), axis=1, keepdims=True)
        pred_loss = jnp.sum(lz - tgt) / G
        focal_loss = fs_ref[...][0, 0] / N
        loss_ref[...] = jnp.full((1, 1), pred_loss + focal_loss + ssl_loss)


def _tc_post(hpk, agc_pk, agn_pk, nh_pk, embsel, w1, w2, wb, wo, wob, nw, fs):
    nb = 10
    rb = N // nb
    VP = wo.shape[1]
    cst = lambda i: (0, 0)
    lo = lambda i: (i, 0)
    hi = lambda i: (nb + i, 0)
    return pl.pallas_call(
        _tc_post_body,
        grid=(nb,),
        in_specs=[
            pl.BlockSpec((rb, HH), lo),
            pl.BlockSpec((rb, HH), hi),
            pl.BlockSpec((rb, HH), lo),
            pl.BlockSpec((rb, HH), hi),
            pl.BlockSpec((rb, HH), lo),
            pl.BlockSpec((rb, HH), hi),
            pl.BlockSpec((G, HH), cst),
            pl.BlockSpec((G, HH), lambda i: (1, 0)),
            pl.BlockSpec((G, H), cst),
            pl.BlockSpec((H, H), cst),
            pl.BlockSpec((H, H), cst),
            pl.BlockSpec((1, H), cst),
            pl.BlockSpec((H, VP), cst),
            pl.BlockSpec((1, VP), cst),
            pl.BlockSpec((G, 1), cst),
            pl.BlockSpec((1, 1), cst),
        ],
        out_specs=pl.BlockSpec((1, 1), cst),
        out_shape=jax.ShapeDtypeStruct((1, 1), jnp.float32),
        scratch_shapes=[
            pltpu.SMEM((2,), jnp.float32),
            pltpu.VMEM((1, H), jnp.float32),
            pltpu.VMEM((1, H), jnp.float32),
        ],
    )(hpk, hpk, agc_pk, agc_pk, agn_pk, agn_pk, nh_pk, nh_pk,
      embsel, w1, w2, wb, wo, wob, nw, fs)


def kernel(ligand_pos, ligand_atom_feature, noise, edge_index, batch_ligand,
           current_atoms, current_atoms_batch, current_wid, next_wid,
           ligand_frontier, W_emb, b_emb, emb_table, W_w, W_b, Wo_w, Wo_b,
           focal_w, focal_b):
    f32 = jnp.float32
    hpk, fsum = _tc_pre(
        ligand_atom_feature,
        W_emb.reshape(D, 2, HH).transpose(1, 0, 2),
        b_emb.reshape(2, 1, HH),
        focal_w.reshape(2, 1, HH),
        focal_b.reshape(1, 1),
        ligand_frontier.reshape(N, 1))

    posn = ligand_pos + 0.3 * noise
    px, py, pz = ligand_pos[:, 0], ligand_pos[:, 1], ligand_pos[:, 2]
    qx, qy, qz = posn[:, 0], posn[:, 1], posn[:, 2]
    src = edge_index[0]
    dst = edge_index[1]
    zeros = jnp.zeros((RPT, HH), f32)

    wc, wn = _sc_weights(px, py, pz, qx, qy, qz, src, dst)
    sdw = jnp.stack([
        src.reshape(E // C, C),
        dst.reshape(E // C, C),
        jax.lax.bitcast_convert_type(wc, jnp.int32).reshape(E // C, C),
        jax.lax.bitcast_convert_type(wn, jnp.int32).reshape(E // C, C),
    ], axis=1)
    aggc_pk, aggn_pk, nh_pk, emb_sel = _sc_edges(
        hpk, sdw,
        current_atoms, current_atoms_batch, current_wid, emb_table, zeros)

    VP = 512
    wo_pad = jnp.zeros((H, VP), f32).at[:, :V].set(Wo_w)
    wob_pad = jnp.full((1, VP), -1e30, f32).at[0, :V].set(Wo_b)

    loss = _tc_post(hpk, aggc_pk, aggn_pk, nh_pk, emb_sel,
                    W_w[:H], W_w[H:], W_b.reshape(1, H), wo_pad, wob_pad,
                    next_wid.reshape(G, 1).astype(jnp.int32), fsum)
    return loss[0, 0]
